# baseline (speedup 1.0000x reference)
import functools

import jax
import jax.numpy as jnp
from jax import lax
from jax.experimental import pallas as pl
from jax.experimental.pallas import tpu as pltpu
from jax.experimental.pallas import tpu_sc as plsc

MAX_STEPS_K = 32
NBINS = MAX_STEPS_K + 1
NCOMBO = 4 * NBINS
ROWW = NCOMBO + 1
LANES = 16
HIST_WORDS = 2 * LANES * ROWW
_UNROLL = 8


def _sc_hist_kernel(chunk, tt_hbm, rs_hbm, hs_hbm, out_hbm,
                    tt_v, rs_v, hs_v, hist_v, sem):
  wid = lax.axis_index("s") * 2 + lax.axis_index("c")
  base = wid * chunk

  cp_tt = pltpu.async_copy(tt_hbm.at[pl.ds(base, chunk)], tt_v, sem)
  cp_rs = pltpu.async_copy(rs_hbm.at[pl.ds(base, chunk)], rs_v, sem)
  cp_hs = pltpu.async_copy(hs_hbm.at[pl.ds(base, chunk)], hs_v, sem)

  zeros16 = jnp.zeros((LANES,), jnp.float32)

  def zero_body(j, _):
    hist_v[pl.ds(j * LANES, LANES)] = zeros16
    return _

  lax.fori_loop(0, HIST_WORDS // LANES, zero_body, None)

  cp_tt.wait()
  cp_rs.wait()
  cp_hs.wait()

  lane = lax.iota(jnp.int32, LANES)
  row_t = lane * ROWW
  row_p = (lane + LANES) * ROWW
  ones16 = jnp.ones((LANES,), jnp.float32)

  @plsc.parallel_loop(0, chunk // LANES, 1, unroll=_UNROLL)
  def _(i):
    b = i * LANES
    tt = tt_v[pl.ds(b, LANES)]
    rs = rs_v[pl.ds(b, LANES)]
    hs = hs_v[pl.ds(b, LANES)]
    c = tt * NBINS
    plsc.addupdate_scatter(hist_v, [row_t + (c + rs)], ones16)
    plsc.addupdate_scatter(hist_v, [row_p + (c + hs)], ones16)

  pltpu.sync_copy(hist_v, out_hbm.at[pl.ds(wid * HIST_WORDS, HIST_WORDS)])


def _tc_kl_kernel(steps, parts_ref, out_ref):
  x = parts_ref[...]
  nrows = x.shape[0]
  row = lax.broadcasted_iota(jnp.int32, (nrows, ROWW), 0)
  is_true = (row % (2 * LANES)) < LANES
  t = jnp.sum(jnp.where(is_true, x, 0.0), axis=0)
  p = jnp.sum(jnp.where(is_true, 0.0, x), axis=0)

  col = lax.iota(jnp.int32, ROWW)
  bin_ = col % NBINS
  ttype = col // NBINS
  valid = (col < NCOMBO) & (bin_ >= 1) & (bin_ <= steps)
  logt = jnp.log(jnp.where(t > 0.0, t, 1.0))
  elt = jnp.where(valid & (t > 0.0), t * (logt - p), 0.0)

  total = jnp.float32(0.0)
  kls = []
  for tt in range(4):
    kl = jnp.sum(jnp.where(ttype == tt, elt, 0.0)) / jnp.float32(steps)
    kls.append(kl)
    total = total + jnp.float32(0.0)

  ocol = lax.broadcasted_iota(jnp.int32, (1, 128), 1)
  vec = jnp.where(ocol == 0, total, jnp.float32(0.0))
  for i, kl in enumerate(kls):
    vec = jnp.where(ocol == (i + 1), kl, vec)
  out_ref[...] = vec


def kernel(trial_types, p_halts, halt_steps, response_steps):
  (b,) = trial_types.shape
  steps = p_halts.shape[1]

  info = plsc.get_sparse_core_info()
  nw = info.num_cores * info.num_subcores
  chunk = b // nw

  mesh = plsc.VectorSubcoreMesh(core_axis_name="c", subcore_axis_name="s")
  sc_call = pl.kernel(
      functools.partial(_sc_hist_kernel, chunk),
      out_type=jax.ShapeDtypeStruct((nw * HIST_WORDS,), jnp.float32),
      mesh=mesh,
      compiler_params=pltpu.CompilerParams(needs_layout_passes=False),
      scratch_types=[
          pltpu.VMEM((chunk,), jnp.int32),
          pltpu.VMEM((chunk,), jnp.int32),
          pltpu.VMEM((chunk,), jnp.int32),
          pltpu.VMEM((HIST_WORDS,), jnp.float32),
          pltpu.SemaphoreType.DMA,
      ],
  )
  parts = sc_call(trial_types.astype(jnp.int32),
                  response_steps.astype(jnp.int32),
                  halt_steps.astype(jnp.int32))

  out = pl.pallas_call(
      functools.partial(_tc_kl_kernel, steps),
      out_shape=jax.ShapeDtypeStruct((1, 128), jnp.float32),
  )(parts.reshape(nw * 2 * LANES, ROWW))
  return out[0, 0]


# speedup vs baseline: 1.1268x; 1.1268x over previous
import functools

import jax
import jax.numpy as jnp
from jax import lax
from jax.experimental import pallas as pl
from jax.experimental.pallas import tpu as pltpu
from jax.experimental.pallas import tpu_sc as plsc

MAX_STEPS_K = 32
NBINS = MAX_STEPS_K + 1
NCOMBO = 4 * NBINS
LANES = 16
HIST_WORDS = 2 * NCOMBO * LANES
HROWS = HIST_WORDS // 128
_UNROLL = 4


def _sc_hist_kernel(chunk, tt_hbm, rs_hbm, hs_hbm, out_hbm,
                    tt_v, rs_v, hs_v, hist_v, shared_v, rowidx_v, sem):
  cid = lax.axis_index("c")
  sid = lax.axis_index("s")
  wid = sid * 2 + cid
  base = wid * chunk

  cp_tt = pltpu.async_copy(tt_hbm.at[pl.ds(base, chunk)], tt_v, sem)
  cp_rs = pltpu.async_copy(rs_hbm.at[pl.ds(base, chunk)], rs_v, sem)
  cp_hs = pltpu.async_copy(hs_hbm.at[pl.ds(base, chunk)], hs_v, sem)

  lane = lax.iota(jnp.int32, LANES)
  zeros16 = jnp.zeros((LANES,), jnp.float32)
  ones16 = jnp.ones((LANES,), jnp.float32)

  rowidx_v[pl.ds(0, LANES)] = lane
  rowidx_v[pl.ds(LANES, LANES)] = lane + LANES
  plsc.store_scatter(rowidx_v, [lane + 2 * LANES], lane + 2 * LANES,
                     mask=lane + 2 * LANES < HROWS)

  def zero_body(r, _):
    for j in range(8):
      hist_v[r, pl.ds(j * LANES, LANES)] = zeros16
    return _

  lax.fori_loop(0, HROWS, zero_body, None)

  cp_tt.wait()
  cp_rs.wait()
  cp_hs.wait()

  cbase = lane

  @plsc.parallel_loop(0, chunk // LANES, 1, unroll=_UNROLL)
  def _(i):
    b = i * LANES
    tt = tt_v[pl.ds(b, LANES)]
    rs = rs_v[pl.ds(b, LANES)]
    hs = hs_v[pl.ds(b, LANES)]
    c = tt * (NBINS * LANES) + cbase
    f_t = c + rs * LANES
    f_p = c + hs * LANES + NCOMBO * LANES
    plsc.addupdate_scatter(
        hist_v, [lax.shift_right_logical(f_t, 7), lax.bitwise_and(f_t, 127)],
        ones16)
    plsc.addupdate_scatter(
        hist_v, [lax.shift_right_logical(f_p, 7), lax.bitwise_and(f_p, 127)],
        ones16)

  @pl.when(sid == 0)
  def _():
    pltpu.sync_copy(hist_v, shared_v)

  plsc.subcore_barrier()

  @pl.when(sid != 0)
  def _():
    pltpu.async_copy(hist_v, shared_v.at[rowidx_v], sem, add=True).wait()

  plsc.subcore_barrier()

  @pl.when(sid == 0)
  def _():
    pltpu.sync_copy(shared_v, out_hbm.at[cid])


def _tc_kl_kernel(steps, parts_ref, out_ref):
  x = parts_ref[...]
  h = jnp.sum(x, axis=2)
  t = h[0]
  p = h[1]

  col = lax.iota(jnp.int32, NCOMBO)
  bin_ = col % NBINS
  valid = (bin_ >= 1) & (bin_ <= steps)
  ttype = col // NBINS
  logt = jnp.log(jnp.where(t > 0.0, t, 1.0))
  elt = jnp.where(valid & (t > 0.0), t * (logt - p), 0.0)

  total = jnp.float32(0.0)
  kls = []
  for tt in range(4):
    kl = jnp.sum(jnp.where(ttype == tt, elt, 0.0)) / jnp.float32(steps)
    kls.append(kl)
    total = total + jnp.float32(0.0)

  ocol = lax.broadcasted_iota(jnp.int32, (1, 128), 1)
  vec = jnp.where(ocol == 0, total, jnp.float32(0.0))
  for i, kl in enumerate(kls):
    vec = jnp.where(ocol == (i + 1), kl, vec)
  out_ref[...] = vec


def kernel(trial_types, p_halts, halt_steps, response_steps):
  (b,) = trial_types.shape
  steps = p_halts.shape[1]

  info = plsc.get_sparse_core_info()
  nw = info.num_cores * info.num_subcores
  chunk = b // nw

  mesh = plsc.VectorSubcoreMesh(core_axis_name="c", subcore_axis_name="s")
  sc_call = pl.kernel(
      functools.partial(_sc_hist_kernel, chunk),
      out_type=jax.ShapeDtypeStruct((info.num_cores, HROWS, 128), jnp.float32),
      mesh=mesh,
      compiler_params=pltpu.CompilerParams(needs_layout_passes=False),
      scratch_types=[
          pltpu.VMEM((chunk,), jnp.int32),
          pltpu.VMEM((chunk,), jnp.int32),
          pltpu.VMEM((chunk,), jnp.int32),
          pltpu.VMEM((HROWS, 128), jnp.float32),
          pltpu.VMEM_SHARED((HROWS, 128), jnp.float32),
          pltpu.VMEM((HROWS,), jnp.int32),
          pltpu.SemaphoreType.DMA,
      ],
  )
  parts = sc_call(trial_types.astype(jnp.int32),
                  response_steps.astype(jnp.int32),
                  halt_steps.astype(jnp.int32))

  arr = parts.reshape(info.num_cores, 2, NCOMBO, LANES)
  arr = arr.transpose(1, 2, 0, 3).reshape(2, NCOMBO, info.num_cores * LANES)

  out = pl.pallas_call(
      functools.partial(_tc_kl_kernel, steps),
      out_shape=jax.ShapeDtypeStruct((1, 128), jnp.float32),
  )(arr)
  return out[0, 0]
